# baseline (speedup 1.0000x reference)
import functools

import jax
import jax.numpy as jnp
from jax import lax
from jax.experimental import pallas as pl
from jax.experimental.pallas import tpu as pltpu
from jax.experimental.pallas import tpu_sc as plsc

N = 10000
D = 128
EPS = 1e-5

NC = 2
NS = 16
NW = NC * NS
NPAD = 10112
ROWS_PER_TILE = NPAD // NS
EB = 128
DEG_GROUP = 8


@functools.cache
def _sc_mesh():
    return plsc.VectorSubcoreMesh(
        core_axis_name="c", subcore_axis_name="s", num_cores=NC, num_subcores=NS
    )


def _deg_body(dst_hbm, ones_hbm, zeros_hbm, out_hbm, dstall, onesv, deg_sp, sem):
    c = lax.axis_index("c")
    s = lax.axis_index("s")
    wid = s * NC + c
    nchunks = dst_hbm.shape[1]
    pltpu.sync_copy(zeros_hbm, deg_sp.at[pl.ds(s * ROWS_PER_TILE, ROWS_PER_TILE)])
    pltpu.sync_copy(ones_hbm, onesv)
    pltpu.sync_copy(dst_hbm.at[wid], dstall)
    plsc.subcore_barrier()

    def group(k, carry):
        base = k * DEG_GROUP
        fired = [
            pltpu.async_copy(onesv, deg_sp.at[dstall.at[base + b]], sem, add=True)
            for b in range(DEG_GROUP)
        ]
        for d in fired:
            d.wait()
        return carry

    lax.fori_loop(0, nchunks // DEG_GROUP, group, 0)
    plsc.subcore_barrier()
    pltpu.sync_copy(
        deg_sp.at[pl.ds(s * ROWS_PER_TILE, ROWS_PER_TILE)],
        out_hbm.at[c, pl.ds(s * ROWS_PER_TILE, ROWS_PER_TILE)],
    )


DEG_W = D


def _sc_degree(dst3, ones, zeros):
    nchunks = dst3.shape[1]
    return pl.kernel(
        _deg_body,
        out_type=jax.ShapeDtypeStruct((NC, NPAD, DEG_W), jnp.float32),
        mesh=_sc_mesh(),
        scratch_types=[
            pltpu.VMEM((nchunks, EB), jnp.int32),
            pltpu.VMEM((EB, DEG_W), jnp.float32),
            pltpu.VMEM_SHARED((NPAD, DEG_W), jnp.float32),
            pltpu.SemaphoreType.DMA,
        ],
    )(dst3, ones, zeros)


def _agg_body(src_hbm, dst_hbm, hs_hbm, zeros_hbm, out_hbm,
              src0, src1, dst0, dst1, rows0, rows1, agg_sp,
              semg0, semg1, semi0, semi1):
    c = lax.axis_index("c")
    s = lax.axis_index("s")
    nchunks = src_hbm.shape[1]

    @pl.when(c == 0)
    def _():
        pltpu.sync_copy(zeros_hbm, agg_sp.at[pl.ds(s * ROWS_PER_TILE, ROWS_PER_TILE)])
        pltpu.sync_copy(src_hbm.at[s, 0], src0)
        pltpu.sync_copy(dst_hbm.at[s, 0], dst0)
        pltpu.async_copy(src_hbm.at[s, 1], src1, semi1)
        pltpu.async_copy(dst_hbm.at[s, 1], dst1, semi1)
        plsc.subcore_barrier()

        dummy_rows = hs_hbm.at[pl.ds(0, EB)]
        dummy_idx = src_hbm.at[0, 0]
        pltpu.async_copy(hs_hbm.at[src0], rows0, semg0)

        def even_half(g, srcA, srcB, dstA, dstB, rowsA, rowsB,
                      semgA, semgB, semiA, semiB):
            pltpu.make_async_copy(dummy_rows, rowsA, semgA).wait()

            @pl.when(g + 1 < nchunks)
            def _():
                pltpu.make_async_copy(dummy_idx, srcB, semiB).wait()
                pltpu.make_async_copy(dummy_idx, dstB, semiB).wait()
                pltpu.async_copy(hs_hbm.at[srcB], rowsB, semgB)

            pltpu.sync_copy(rowsA, agg_sp.at[dstA], add=True)

            @pl.when(g + 2 < nchunks)
            def _():
                pltpu.async_copy(src_hbm.at[s, g + 2], srcA, semiA)
                pltpu.async_copy(dst_hbm.at[s, g + 2], dstA, semiA)

        def step(g, carry):
            @pl.when(g % 2 == 0)
            def _():
                even_half(g, src0, src1, dst0, dst1, rows0, rows1,
                          semg0, semg1, semi0, semi1)

            @pl.when(g % 2 == 1)
            def _():
                even_half(g, src1, src0, dst1, dst0, rows1, rows0,
                          semg1, semg0, semi1, semi0)

            return carry

        lax.fori_loop(0, nchunks, step, 0)
        plsc.subcore_barrier()
        pltpu.sync_copy(
            agg_sp.at[pl.ds(s * ROWS_PER_TILE, ROWS_PER_TILE)],
            out_hbm.at[pl.ds(s * ROWS_PER_TILE, ROWS_PER_TILE)],
        )


def _sc_aggregate(src2, dst2, hs, zeros):
    nchunks = src2.shape[1]
    return pl.kernel(
        _agg_body,
        out_type=jax.ShapeDtypeStruct((NPAD, D), jnp.float32),
        mesh=_sc_mesh(),
        scratch_types=[
            pltpu.VMEM((EB,), jnp.int32),
            pltpu.VMEM((EB,), jnp.int32),
            pltpu.VMEM((EB,), jnp.int32),
            pltpu.VMEM((EB,), jnp.int32),
            pltpu.VMEM((EB, D), jnp.float32),
            pltpu.VMEM((EB, D), jnp.float32),
            pltpu.VMEM_SHARED((NPAD, D), jnp.float32),
            pltpu.SemaphoreType.DMA,
            pltpu.SemaphoreType.DMA,
            pltpu.SemaphoreType.DMA,
            pltpu.SemaphoreType.DMA,
        ],
    )(src2, dst2, hs, zeros)


BN = 2000


def _mm_body(x_ref, w_ref, d0_ref, d1_ref, hs_ref):
    dinv = lax.rsqrt(d0_ref[...] + d1_ref[...] + 1.0)
    h = jnp.dot(x_ref[...], w_ref[...], preferred_element_type=jnp.float32)
    hs_ref[...] = h * dinv


def _tc_matmul_prescale(x, w, d0, d1):
    grid = (N // BN,)
    return pl.pallas_call(
        _mm_body,
        grid=grid,
        in_specs=[
            pl.BlockSpec((BN, D), lambda i: (i, 0)),
            pl.BlockSpec((D, D), lambda i: (0, 0)),
            pl.BlockSpec((BN, 1), lambda i: (i, 0)),
            pl.BlockSpec((BN, 1), lambda i: (i, 0)),
        ],
        out_specs=pl.BlockSpec((BN, D), lambda i: (i, 0)),
        out_shape=jax.ShapeDtypeStruct((N, D), jnp.float32),
    )(x, w, d0, d1)


def _ln(t, g, be):
    mu = jnp.mean(t, axis=1, keepdims=True)
    xc = t - mu
    var = jnp.mean(xc * xc, axis=1, keepdims=True)
    return xc * lax.rsqrt(var + EPS) * g + be


def _combine_mm_body(p0_ref, hs_ref, d0_ref, d1_ref, b_ref, g_ref,
                     be_ref, w_ref, out_ref):
    dinv = lax.rsqrt(d0_ref[...] + d1_ref[...] + 1.0)
    agg = (p0_ref[...] + hs_ref[...]) * dinv + b_ref[...]
    y = _ln(jnp.tanh(agg), g_ref[...], be_ref[...])
    out_ref[...] = jnp.dot(y, w_ref[...], preferred_element_type=jnp.float32) * dinv


def _combine_body(p0_ref, hs_ref, d0_ref, d1_ref, b_ref, g_ref,
                  be_ref, out_ref):
    dinv = lax.rsqrt(d0_ref[...] + d1_ref[...] + 1.0)
    agg = (p0_ref[...] + hs_ref[...]) * dinv + b_ref[...]
    out_ref[...] = _ln(jnp.tanh(agg), g_ref[...], be_ref[...])


def _tc_combine(p0, hs, d0, d1, b, g, be, w_next=None):
    grid = (N // BN,)
    row_spec = pl.BlockSpec((BN, D), lambda i: (i, 0))
    col_spec = pl.BlockSpec((BN, 1), lambda i: (i, 0))
    vec_spec = pl.BlockSpec((1, D), lambda i: (0, 0))
    in_specs = [row_spec, row_spec, col_spec, col_spec,
                vec_spec, vec_spec, vec_spec]
    args = [p0, hs, d0, d1, b, g, be]
    body = _combine_body
    if w_next is not None:
        in_specs.append(pl.BlockSpec((D, D), lambda i: (0, 0)))
        args.append(w_next)
        body = _combine_mm_body
    return pl.pallas_call(
        body,
        grid=grid,
        in_specs=in_specs,
        out_specs=row_spec,
        out_shape=jax.ShapeDtypeStruct((N, D), jnp.float32),
    )(*args)


def _pad_edges(src, dst, total):
    epad = total - src.shape[0]
    if epad:
        src = jnp.concatenate([src, jnp.zeros((epad,), jnp.int32)])
        trash = N + (jnp.arange(epad, dtype=jnp.int32) % (NPAD - N))
        dst = jnp.concatenate([dst, trash])
    return src, dst


def kernel(x, edge_index, W1, b1, g1, be1, W2, b2, g2, be2):
    E = edge_index.shape[1]
    src = edge_index[0]
    dst = edge_index[1]

    nchunks = -(-E // (NW * EB))
    nchunks = -(-nchunks // DEG_GROUP) * DEG_GROUP
    dsrc, ddst = _pad_edges(src, dst, NW * EB * nchunks)
    dst3e = ddst.reshape(NW, nchunks, EB)

    nagg = -(-E // (NS * EB))
    nagg += nagg % 2
    asrc, adst = _pad_edges(src, dst, NS * EB * nagg)
    src2 = asrc.reshape(NS, nagg, EB)
    dst2 = adst.reshape(NS, nagg, EB)

    onesW = jnp.ones((EB, DEG_W), jnp.float32)
    zerosW = jnp.zeros((ROWS_PER_TILE, DEG_W), jnp.float32)
    zerosD = jnp.zeros((ROWS_PER_TILE, D), jnp.float32)

    deg = _sc_degree(dst3e, onesW, zerosW)
    d0 = deg[0, :N, :1]
    d1 = deg[1, :N, :1]

    b1r, g1r, be1r = b1.reshape(1, D), g1.reshape(1, D), be1.reshape(1, D)
    b2r, g2r, be2r = b2.reshape(1, D), g2.reshape(1, D), be2.reshape(1, D)

    hs1 = _tc_matmul_prescale(x, W1, d0, d1)
    p1 = _sc_aggregate(src2, dst2, hs1, zerosD)
    hs2 = _tc_combine(p1[:N], hs1, d0, d1, b1r, g1r, be1r, W2)
    p2 = _sc_aggregate(src2, dst2, hs2, zerosD)
    out = _tc_combine(p2[:N], hs2, d0, d1, b2r, g2r, be2r)
    return out


# speedup vs baseline: 1.2178x; 1.2178x over previous
import functools

import jax
import jax.numpy as jnp
from jax import lax
from jax.experimental import pallas as pl
from jax.experimental.pallas import tpu as pltpu
from jax.experimental.pallas import tpu_sc as plsc

N = 10000
D = 128
EPS = 1e-5

NC = 2
NS = 16
NW = NC * NS
NPAD = 10112
ROWS_PER_TILE = NPAD // NS
EB = 128
DEG_GROUP = 8


@functools.cache
def _sc_mesh():
    return plsc.VectorSubcoreMesh(
        core_axis_name="c", subcore_axis_name="s", num_cores=NC, num_subcores=NS
    )


def _deg_body(dst_hbm, ones_hbm, zeros_hbm, out_hbm, dstall, onesv, deg_sp, sem):
    c = lax.axis_index("c")
    s = lax.axis_index("s")
    wid = s * NC + c
    nchunks = dst_hbm.shape[1]
    pltpu.sync_copy(zeros_hbm, deg_sp.at[pl.ds(s * ROWS_PER_TILE, ROWS_PER_TILE)])
    pltpu.sync_copy(ones_hbm, onesv)
    pltpu.sync_copy(dst_hbm.at[wid], dstall)
    plsc.subcore_barrier()

    def group(k, carry):
        base = k * DEG_GROUP
        fired = [
            pltpu.async_copy(onesv, deg_sp.at[dstall.at[base + b]], sem, add=True)
            for b in range(DEG_GROUP)
        ]
        for d in fired:
            d.wait()
        return carry

    lax.fori_loop(0, nchunks // DEG_GROUP, group, 0)
    plsc.subcore_barrier()
    pltpu.sync_copy(
        deg_sp.at[pl.ds(s * ROWS_PER_TILE, ROWS_PER_TILE)],
        out_hbm.at[c, pl.ds(s * ROWS_PER_TILE, ROWS_PER_TILE)],
    )


DEG_W = D


def _sc_degree(dst3, ones, zeros):
    nchunks = dst3.shape[1]
    return pl.kernel(
        _deg_body,
        out_type=jax.ShapeDtypeStruct((NC, NPAD, DEG_W), jnp.float32),
        mesh=_sc_mesh(),
        scratch_types=[
            pltpu.VMEM((nchunks, EB), jnp.int32),
            pltpu.VMEM((EB, DEG_W), jnp.float32),
            pltpu.VMEM_SHARED((NPAD, DEG_W), jnp.float32),
            pltpu.SemaphoreType.DMA,
        ],
    )(dst3, ones, zeros)


def _agg_body(src_hbm, dst_hbm, hs_hbm, zeros_hbm, out_hbm,
              src0, src1, dst0, dst1, rows0, rows1, agg_sp,
              semg0, semg1, semi0, semi1):
    c = lax.axis_index("c")
    s = lax.axis_index("s")
    nchunks = src_hbm.shape[1]

    @pl.when(c == 0)
    def _():
        pltpu.sync_copy(zeros_hbm, agg_sp.at[pl.ds(s * ROWS_PER_TILE, ROWS_PER_TILE)])
        pltpu.sync_copy(src_hbm.at[s, 0], src0)
        pltpu.sync_copy(dst_hbm.at[s, 0], dst0)
        pltpu.async_copy(src_hbm.at[s, 1], src1, semi1)
        pltpu.async_copy(dst_hbm.at[s, 1], dst1, semi1)
        plsc.subcore_barrier()

        dummy_rows = hs_hbm.at[pl.ds(0, EB)]
        dummy_idx = src_hbm.at[0, 0]
        pltpu.async_copy(hs_hbm.at[src0], rows0, semg0)

        def even_half(g, srcA, srcB, dstA, dstB, rowsA, rowsB,
                      semgA, semgB, semiA, semiB):
            pltpu.make_async_copy(dummy_rows, rowsA, semgA).wait()

            @pl.when(g + 1 < nchunks)
            def _():
                pltpu.make_async_copy(dummy_idx, srcB, semiB).wait()
                pltpu.make_async_copy(dummy_idx, dstB, semiB).wait()
                pltpu.async_copy(hs_hbm.at[srcB], rowsB, semgB)

            pltpu.sync_copy(rowsA, agg_sp.at[dstA], add=True)

            @pl.when(g + 2 < nchunks)
            def _():
                pltpu.async_copy(src_hbm.at[s, g + 2], srcA, semiA)
                pltpu.async_copy(dst_hbm.at[s, g + 2], dstA, semiA)

        def step(g, carry):
            @pl.when(g % 2 == 0)
            def _():
                even_half(g, src0, src1, dst0, dst1, rows0, rows1,
                          semg0, semg1, semi0, semi1)

            @pl.when(g % 2 == 1)
            def _():
                even_half(g, src1, src0, dst1, dst0, rows1, rows0,
                          semg1, semg0, semi1, semi0)

            return carry

        lax.fori_loop(0, nchunks, step, 0)
        plsc.subcore_barrier()
        pltpu.sync_copy(
            agg_sp.at[pl.ds(s * ROWS_PER_TILE, ROWS_PER_TILE)],
            out_hbm.at[pl.ds(s * ROWS_PER_TILE, ROWS_PER_TILE)],
        )


def _sc_aggregate(src2, dst2, hs, zeros):
    nchunks = src2.shape[1]
    return pl.kernel(
        _agg_body,
        out_type=jax.ShapeDtypeStruct((NPAD, D), jnp.float32),
        mesh=_sc_mesh(),
        scratch_types=[
            pltpu.VMEM((EB,), jnp.int32),
            pltpu.VMEM((EB,), jnp.int32),
            pltpu.VMEM((EB,), jnp.int32),
            pltpu.VMEM((EB,), jnp.int32),
            pltpu.VMEM((EB, D), jnp.float32),
            pltpu.VMEM((EB, D), jnp.float32),
            pltpu.VMEM_SHARED((NPAD, D), jnp.float32),
            pltpu.SemaphoreType.DMA,
            pltpu.SemaphoreType.DMA,
            pltpu.SemaphoreType.DMA,
            pltpu.SemaphoreType.DMA,
        ],
    )(src2, dst2, hs, zeros)


BN = 2000


def _mm_body(x_ref, w_ref, d0_ref, d1_ref, hs_ref):
    dinv = lax.rsqrt(d0_ref[...] + d1_ref[...] + 1.0)
    h = jnp.dot(x_ref[...], w_ref[...], preferred_element_type=jnp.float32)
    hs_ref[...] = h * dinv


def _tc_matmul_prescale(x, w, d0, d1):
    grid = (N // BN,)
    return pl.pallas_call(
        _mm_body,
        grid=grid,
        in_specs=[
            pl.BlockSpec((BN, D), lambda i: (i, 0)),
            pl.BlockSpec((D, D), lambda i: (0, 0)),
            pl.BlockSpec((BN, 1), lambda i: (i, 0)),
            pl.BlockSpec((BN, 1), lambda i: (i, 0)),
        ],
        out_specs=pl.BlockSpec((BN, D), lambda i: (i, 0)),
        out_shape=jax.ShapeDtypeStruct((N, D), jnp.float32),
    )(x, w, d0, d1)


def _ln(t, g, be):
    mu = jnp.mean(t, axis=1, keepdims=True)
    xc = t - mu
    var = jnp.mean(xc * xc, axis=1, keepdims=True)
    return xc * lax.rsqrt(var + EPS) * g + be


def _combine_mm_body(p0_ref, hs_ref, d0_ref, d1_ref, b_ref, g_ref,
                     be_ref, w_ref, out_ref):
    dinv = lax.rsqrt(d0_ref[...] + d1_ref[...] + 1.0)
    agg = (p0_ref[...] + hs_ref[...]) * dinv + b_ref[...]
    y = _ln(jnp.tanh(agg), g_ref[...], be_ref[...])
    out_ref[...] = jnp.dot(y, w_ref[...], preferred_element_type=jnp.float32) * dinv


def _combine_body(p0_ref, hs_ref, d0_ref, d1_ref, b_ref, g_ref,
                  be_ref, out_ref):
    dinv = lax.rsqrt(d0_ref[...] + d1_ref[...] + 1.0)
    agg = (p0_ref[...] + hs_ref[...]) * dinv + b_ref[...]
    out_ref[...] = _ln(jnp.tanh(agg), g_ref[...], be_ref[...])


def _tc_combine(p0, hs, d0, d1, b, g, be, w_next=None):
    grid = (N // BN,)
    row_spec = pl.BlockSpec((BN, D), lambda i: (i, 0))
    col_spec = pl.BlockSpec((BN, 1), lambda i: (i, 0))
    vec_spec = pl.BlockSpec((1, D), lambda i: (0, 0))
    in_specs = [row_spec, row_spec, col_spec, col_spec,
                vec_spec, vec_spec, vec_spec]
    args = [p0, hs, d0, d1, b, g, be]
    body = _combine_body
    if w_next is not None:
        in_specs.append(pl.BlockSpec((D, D), lambda i: (0, 0)))
        args.append(w_next)
        body = _combine_mm_body
    return pl.pallas_call(
        body,
        grid=grid,
        in_specs=in_specs,
        out_specs=row_spec,
        out_shape=jax.ShapeDtypeStruct((N, D), jnp.float32),
    )(*args)


def _pad_edges(src, dst, total):
    epad = total - src.shape[0]
    if epad:
        src = jnp.concatenate([src, jnp.zeros((epad,), jnp.int32)])
        trash = N + (jnp.arange(epad, dtype=jnp.int32) % (NPAD - N))
        dst = jnp.concatenate([dst, trash])
    return src, dst


def kernel(x, edge_index, W1, b1, g1, be1, W2, b2, g2, be2):
    E = edge_index.shape[1]
    src = edge_index[0]
    dst = edge_index[1]

    nchunks = -(-E // (NW * EB))
    nchunks = -(-nchunks // DEG_GROUP) * DEG_GROUP
    dsrc, ddst = _pad_edges(src, dst, NW * EB * nchunks)
    dst3e = ddst.reshape(NW, nchunks, EB)

    nagg = -(-E // (NS * EB))
    asrc, adst = _pad_edges(src, dst, NS * EB * nagg)
    src2 = asrc.reshape(NS, nagg, EB)
    dst2 = adst.reshape(NS, nagg, EB)

    onesW = jnp.ones((EB, DEG_W), jnp.float32)
    zerosW = jnp.zeros((ROWS_PER_TILE, DEG_W), jnp.float32)
    zerosD = jnp.zeros((ROWS_PER_TILE, D), jnp.float32)

    deg = _sc_degree(dst3e, onesW, zerosW)
    d0 = deg[0, :N, :1]
    d1 = deg[1, :N, :1]

    b1r, g1r, be1r = b1.reshape(1, D), g1.reshape(1, D), be1.reshape(1, D)
    b2r, g2r, be2r = b2.reshape(1, D), g2.reshape(1, D), be2.reshape(1, D)

    hs1 = _tc_matmul_prescale(x, W1, d0, d1)
    p1 = _sc_aggregate(src2, dst2, hs1, zerosD)
    hs2 = _tc_combine(p1[:N], hs1, d0, d1, b1r, g1r, be1r, W2)
    p2 = _sc_aggregate(src2, dst2, hs2, zerosD)
    out = _tc_combine(p2[:N], hs2, d0, d1, b2r, g2r, be2r)
    return out
